# baseline (speedup 1.0000x reference)
import jax
import jax.numpy as jnp
from jax.experimental import pallas as pl
from jax.experimental.pallas import tpu as pltpu

_K = 512
_D = 64
_NUM_CLASSES = 60
_H = 32
_W = 32
_HB = 8
_NB = _HB * _W


def _vq_body(c_ref, z_ref, cb_ref, out_ref):
    z = z_ref[0].reshape(_D, _NB)
    sub = cb_ref[0]
    e_sq = jnp.sum(sub * sub, axis=1, keepdims=True)
    cross = jax.lax.dot_general(
        sub, z, (((1,), (0,)), ((), ())),
        preferred_element_type=jnp.float32)
    dist = e_sq - 2.0 * cross
    minv = jnp.min(dist, axis=0, keepdims=True)
    iota = jax.lax.broadcasted_iota(jnp.int32, (_K, _NB), 0)
    idx = jnp.min(jnp.where(dist == minv, iota, _K), axis=0, keepdims=True)
    onehot = (iota == idx).astype(jnp.float32)
    quant = jax.lax.dot_general(
        onehot, sub, (((0,), (0,)), ((), ())),
        preferred_element_type=jnp.float32)
    out_ref[0] = quant.reshape(_HB, _W, _D)


def kernel(z_e_x, c, emb_weight):
    B = z_e_x.shape[0]
    codebooks = emb_weight.reshape(_NUM_CLASSES, _K, _D)
    grid_spec = pltpu.PrefetchScalarGridSpec(
        num_scalar_prefetch=1,
        grid=(B, _H // _HB),
        in_specs=[
            pl.BlockSpec((1, _D, _HB, _W), lambda b, n, c_ref: (b, 0, n, 0)),
            pl.BlockSpec((1, _K, _D), lambda b, n, c_ref: (c_ref[b], 0, 0)),
        ],
        out_specs=pl.BlockSpec((1, _HB, _W, _D), lambda b, n, c_ref: (b, n, 0, 0)),
    )
    return pl.pallas_call(
        _vq_body,
        grid_spec=grid_spec,
        out_shape=jax.ShapeDtypeStruct((B, _H, _W, _D), jnp.float32),
    )(c, z_e_x, codebooks)


# speedup vs baseline: 1.2652x; 1.2652x over previous
import jax
import jax.numpy as jnp
from jax.experimental import pallas as pl
from jax.experimental.pallas import tpu as pltpu

_K = 512
_D = 64
_NUM_CLASSES = 60
_N = 1024
_NB = 256


def _vq_body(c_ref, z_ref, cbt_ref, out_ref):
    z = z_ref[0]
    subt = cbt_ref[:, 0, 0, :]
    e_sq = jnp.sum(subt * subt, axis=0, keepdims=True)
    cross = jax.lax.dot_general(
        z, subt, (((1,), (0,)), ((), ())),
        preferred_element_type=jnp.float32)
    dist = e_sq - 2.0 * cross
    minv = jnp.min(dist, axis=1, keepdims=True)
    iota = jax.lax.broadcasted_iota(jnp.int32, (_NB, _K), 1)
    idx = jnp.min(jnp.where(dist == minv, iota, _K), axis=1, keepdims=True)
    onehot = (iota == idx).astype(jnp.float32)
    quant = jax.lax.dot_general(
        onehot, subt, (((1,), (1,)), ((), ())),
        preferred_element_type=jnp.float32)
    out_ref[0] = quant


def kernel(z_e_x, c, emb_weight):
    B = z_e_x.shape[0]
    zf = jnp.transpose(z_e_x, (0, 2, 3, 1)).reshape(B, _N, _D)
    cbt = jnp.transpose(emb_weight, (1, 0)).reshape(_D, _NUM_CLASSES, 1, _K)
    grid_spec = pltpu.PrefetchScalarGridSpec(
        num_scalar_prefetch=1,
        grid=(B, _N // _NB),
        in_specs=[
            pl.BlockSpec((1, _NB, _D), lambda b, n, c_ref: (b, n, 0)),
            pl.BlockSpec((_D, 1, 1, _K), lambda b, n, c_ref: (0, c_ref[b], 0, 0)),
        ],
        out_specs=pl.BlockSpec((1, _NB, _D), lambda b, n, c_ref: (b, n, 0)),
    )
    out = pl.pallas_call(
        _vq_body,
        grid_spec=grid_spec,
        out_shape=jax.ShapeDtypeStruct((B, _N, _D), jnp.float32),
    )(c, zf, cbt)
    return out.reshape(B, 32, 32, _D)


# speedup vs baseline: 2.2226x; 1.7567x over previous
import jax
import jax.numpy as jnp
from jax.experimental import pallas as pl
from jax.experimental.pallas import tpu as pltpu

_K = 512
_D = 64
_NUM_CLASSES = 60
_N = 1024
_NB = 1024


def _vq_body(c_ref, z_ref, cbt_ref, out_ref):
    z = z_ref[0]
    subt = cbt_ref[:, 0, 0, :]
    e_sq = jnp.sum(subt * subt, axis=0, keepdims=True)
    cross = jax.lax.dot_general(
        z, subt, (((1,), (0,)), ((), ())),
        preferred_element_type=jnp.float32)
    dist = e_sq - 2.0 * cross
    minv = jnp.min(dist, axis=1, keepdims=True)
    iota = jax.lax.broadcasted_iota(jnp.int32, (_NB, _K), 1)
    idx = jnp.min(jnp.where(dist == minv, iota, _K), axis=1, keepdims=True)
    onehot = (iota == idx).astype(jnp.float32)
    quant = jax.lax.dot_general(
        onehot, subt, (((1,), (1,)), ((), ())),
        preferred_element_type=jnp.float32)
    out_ref[0] = quant


def kernel(z_e_x, c, emb_weight):
    B = z_e_x.shape[0]
    zf = jnp.transpose(z_e_x, (0, 2, 3, 1)).reshape(B, _N, _D)
    cbt = jnp.transpose(emb_weight, (1, 0)).reshape(_D, _NUM_CLASSES, 1, _K)
    grid_spec = pltpu.PrefetchScalarGridSpec(
        num_scalar_prefetch=1,
        grid=(B, _N // _NB),
        in_specs=[
            pl.BlockSpec((1, _NB, _D), lambda b, n, c_ref: (b, n, 0)),
            pl.BlockSpec((_D, 1, 1, _K), lambda b, n, c_ref: (0, c_ref[b], 0, 0)),
        ],
        out_specs=pl.BlockSpec((1, _NB, _D), lambda b, n, c_ref: (b, n, 0)),
    )
    out = pl.pallas_call(
        _vq_body,
        grid_spec=grid_spec,
        out_shape=jax.ShapeDtypeStruct((B, _N, _D), jnp.float32),
    )(c, zf, cbt)
    return out.reshape(B, 32, 32, _D)


# speedup vs baseline: 4.2526x; 1.9134x over previous
import jax
import jax.numpy as jnp
from jax.experimental import pallas as pl
from jax.experimental.pallas import tpu as pltpu

_K = 512
_D = 64
_NUM_CLASSES = 60
_N = 1024


def _vq_body(c_ref, z_hbm, cbt_hbm, out_ref, z_buf, cb_buf, z_sem, cb_sem):
    b = pl.program_id(0)
    nb = pl.num_programs(0)
    slot = jax.lax.rem(b, 2)
    nxt = jax.lax.rem(b + 1, 2)

    def _start(i, s):
        pltpu.make_async_copy(z_hbm.at[i], z_buf.at[s], z_sem.at[s]).start()
        pltpu.make_async_copy(
            cbt_hbm.at[:, pl.ds(c_ref[i] * _K, _K)], cb_buf.at[s], cb_sem.at[s]
        ).start()

    @pl.when(b == 0)
    def _():
        _start(0, 0)

    @pl.when(b + 1 < nb)
    def _():
        _start(b + 1, nxt)

    pltpu.make_async_copy(z_hbm.at[b], z_buf.at[slot], z_sem.at[slot]).wait()
    pltpu.make_async_copy(
        cbt_hbm.at[:, pl.ds(c_ref[b] * _K, _K)], cb_buf.at[slot], cb_sem.at[slot]
    ).wait()

    z = z_buf[slot]
    subt = cb_buf[slot]
    e_sq = jnp.sum(subt * subt, axis=0, keepdims=True)
    cross = jax.lax.dot_general(
        z, subt, (((1,), (0,)), ((), ())),
        preferred_element_type=jnp.float32)
    dist = e_sq - 2.0 * cross
    minv = jnp.min(dist, axis=1, keepdims=True)
    iota = jax.lax.broadcasted_iota(jnp.int32, (_N, _K), 1)
    idx = jnp.min(jnp.where(dist == minv, iota, _K), axis=1, keepdims=True)
    onehot = (iota == idx).astype(jnp.float32)
    quant = jax.lax.dot_general(
        onehot, subt, (((1,), (1,)), ((), ())),
        preferred_element_type=jnp.float32)
    out_ref[0] = quant


def kernel(z_e_x, c, emb_weight):
    B = z_e_x.shape[0]
    zf = jnp.transpose(z_e_x, (0, 2, 3, 1)).reshape(B, _N, _D)
    cbt = jnp.transpose(emb_weight, (1, 0))
    zf = pltpu.with_memory_space_constraint(zf, pltpu.MemorySpace.HBM)
    cbt = pltpu.with_memory_space_constraint(cbt, pltpu.MemorySpace.HBM)
    grid_spec = pltpu.PrefetchScalarGridSpec(
        num_scalar_prefetch=1,
        grid=(B,),
        in_specs=[
            pl.BlockSpec(memory_space=pltpu.MemorySpace.HBM),
            pl.BlockSpec(memory_space=pltpu.MemorySpace.HBM),
        ],
        out_specs=pl.BlockSpec((1, _N, _D), lambda b, c_ref: (b, 0, 0)),
        scratch_shapes=[
            pltpu.VMEM((2, _N, _D), jnp.float32),
            pltpu.VMEM((2, _D, _K), jnp.float32),
            pltpu.SemaphoreType.DMA((2,)),
            pltpu.SemaphoreType.DMA((2,)),
        ],
    )
    out = pl.pallas_call(
        _vq_body,
        grid_spec=grid_spec,
        out_shape=jax.ShapeDtypeStruct((B, _N, _D), jnp.float32),
    )(c, zf, cbt)
    return out.reshape(B, 32, 32, _D)


# speedup vs baseline: 4.3066x; 1.0127x over previous
import jax
import jax.numpy as jnp
from jax.experimental import pallas as pl
from jax.experimental.pallas import tpu as pltpu

_K = 512
_D = 64
_NUM_CLASSES = 60
_N = 1024


def _vq_body(c_ref, z_hbm, cbt_hbm, out_ref, z_buf, cb_buf, z_sem, cb_sem):
    b = pl.program_id(0)
    nb = pl.num_programs(0)
    slot = jax.lax.rem(b, 2)
    nxt = jax.lax.rem(b + 1, 2)

    def _start(i, s):
        pltpu.make_async_copy(z_hbm.at[i], z_buf.at[s], z_sem.at[s]).start()
        pltpu.make_async_copy(
            cbt_hbm.at[:, pl.ds(c_ref[i] * _K, _K)], cb_buf.at[s], cb_sem.at[s]
        ).start()

    @pl.when(b == 0)
    def _():
        _start(0, 0)

    @pl.when(b + 1 < nb)
    def _():
        _start(b + 1, nxt)

    pltpu.make_async_copy(z_hbm.at[b], z_buf.at[slot], z_sem.at[slot]).wait()
    pltpu.make_async_copy(
        cbt_hbm.at[:, pl.ds(c_ref[b] * _K, _K)], cb_buf.at[slot], cb_sem.at[slot]
    ).wait()

    z = z_buf[slot]
    subt = cb_buf[slot]
    e_sq = jnp.sum(subt * subt, axis=0, keepdims=True)
    subt2 = -2.0 * subt
    cross2 = jax.lax.dot_general(
        z, subt2, (((1,), (0,)), ((), ())),
        preferred_element_type=jnp.float32)
    dist = cross2 + e_sq
    minv = jnp.min(dist, axis=1, keepdims=True)
    iota = jax.lax.broadcasted_iota(jnp.int32, (_N, _K), 1)
    idx = jnp.min(jnp.where(dist == minv, iota, _K), axis=1, keepdims=True)
    onehot = (iota == idx).astype(jnp.float32)
    quant = jax.lax.dot_general(
        onehot, subt, (((1,), (1,)), ((), ())),
        preferred_element_type=jnp.float32)
    out_ref[0] = quant


def kernel(z_e_x, c, emb_weight):
    B = z_e_x.shape[0]
    zf = jnp.transpose(z_e_x, (0, 2, 3, 1)).reshape(B, _N, _D)
    cbt = jnp.transpose(emb_weight, (1, 0))
    zf = pltpu.with_memory_space_constraint(zf, pltpu.MemorySpace.HBM)
    cbt = pltpu.with_memory_space_constraint(cbt, pltpu.MemorySpace.HBM)
    grid_spec = pltpu.PrefetchScalarGridSpec(
        num_scalar_prefetch=1,
        grid=(B,),
        in_specs=[
            pl.BlockSpec(memory_space=pltpu.MemorySpace.HBM),
            pl.BlockSpec(memory_space=pltpu.MemorySpace.HBM),
        ],
        out_specs=pl.BlockSpec((1, _N, _D), lambda b, c_ref: (b, 0, 0)),
        scratch_shapes=[
            pltpu.VMEM((2, _N, _D), jnp.float32),
            pltpu.VMEM((2, _D, _K), jnp.float32),
            pltpu.SemaphoreType.DMA((2,)),
            pltpu.SemaphoreType.DMA((2,)),
        ],
    )
    out = pl.pallas_call(
        _vq_body,
        grid_spec=grid_spec,
        out_shape=jax.ShapeDtypeStruct((B, _N, _D), jnp.float32),
    )(c, zf, cbt)
    return out.reshape(B, 32, 32, _D)


# speedup vs baseline: 4.7000x; 1.0913x over previous
import jax
import jax.numpy as jnp
from jax.experimental import pallas as pl
from jax.experimental.pallas import tpu as pltpu

_K = 512
_D = 64
_NUM_CLASSES = 60
_N = 1024


def _vq_body(c_ref, z_hbm, cbt_hbm, out_ref, z_buf, cb_buf, z_sem, cb_sem):
    b = pl.program_id(0)
    nb = pl.num_programs(0)
    slot = jax.lax.rem(b, 3)
    nxt = jax.lax.rem(b + 2, 3)

    def _start(i, s):
        pltpu.make_async_copy(z_hbm.at[i], z_buf.at[s], z_sem.at[s]).start()
        pltpu.make_async_copy(
            cbt_hbm.at[:, pl.ds(c_ref[i] * _K, _K)], cb_buf.at[s], cb_sem.at[s]
        ).start()

    @pl.when(b == 0)
    def _():
        _start(0, 0)
        _start(1, 1)

    @pl.when(b + 2 < nb)
    def _():
        _start(b + 2, nxt)

    pltpu.make_async_copy(z_hbm.at[b], z_buf.at[slot], z_sem.at[slot]).wait()
    pltpu.make_async_copy(
        cbt_hbm.at[:, pl.ds(c_ref[b] * _K, _K)], cb_buf.at[slot], cb_sem.at[slot]
    ).wait()

    z = z_buf[slot]
    subt = cb_buf[slot]
    e_sq = jnp.sum(subt * subt, axis=0, keepdims=True)
    subt2 = -2.0 * subt
    cross2 = jax.lax.dot_general(
        z, subt2, (((1,), (0,)), ((), ())),
        preferred_element_type=jnp.float32)
    dist = cross2 + e_sq
    minv = jnp.min(dist, axis=1, keepdims=True)
    iota = jax.lax.broadcasted_iota(jnp.int32, (_N, _K), 1)
    idx = jnp.min(jnp.where(dist == minv, iota, _K), axis=1, keepdims=True)
    onehot = (iota == idx).astype(jnp.float32)
    quant = jax.lax.dot_general(
        onehot, subt, (((1,), (1,)), ((), ())),
        preferred_element_type=jnp.float32)
    out_ref[0] = quant


def kernel(z_e_x, c, emb_weight):
    B = z_e_x.shape[0]
    zf = jnp.transpose(z_e_x, (0, 2, 3, 1)).reshape(B, _N, _D)
    cbt = jnp.transpose(emb_weight, (1, 0))
    zf = pltpu.with_memory_space_constraint(zf, pltpu.MemorySpace.HBM)
    cbt = pltpu.with_memory_space_constraint(cbt, pltpu.MemorySpace.HBM)
    grid_spec = pltpu.PrefetchScalarGridSpec(
        num_scalar_prefetch=1,
        grid=(B,),
        in_specs=[
            pl.BlockSpec(memory_space=pltpu.MemorySpace.HBM),
            pl.BlockSpec(memory_space=pltpu.MemorySpace.HBM),
        ],
        out_specs=pl.BlockSpec((1, _N, _D), lambda b, c_ref: (b, 0, 0)),
        scratch_shapes=[
            pltpu.VMEM((3, _N, _D), jnp.float32),
            pltpu.VMEM((3, _D, _K), jnp.float32),
            pltpu.SemaphoreType.DMA((3,)),
            pltpu.SemaphoreType.DMA((3,)),
        ],
    )
    out = pl.pallas_call(
        _vq_body,
        grid_spec=grid_spec,
        out_shape=jax.ShapeDtypeStruct((B, _N, _D), jnp.float32),
    )(c, zf, cbt)
    return out.reshape(B, 32, 32, _D)
